# Initial kernel scaffold; baseline (speedup 1.0000x reference)
import functools

import jax
import jax.numpy as jnp
from jax import lax
from jax.experimental import pallas as pl
from jax.experimental.pallas import tpu as pltpu
from jax.experimental.pallas import tpu_sc as plsc

NC = 2
NS = 16
NW = NC * NS
L = 16
K = 128


def _matmul_kernel(x_ref, w_ref, o_ref):
    o_ref[...] = jnp.dot(x_ref[...], w_ref[...],
                         preferred_element_type=jnp.float32)


def _combine_kernel(p0_ref, p1_ref, b_ref, o_ref):
    o_ref[...] = p0_ref[...] + p1_ref[...] + b_ref[...]


def _make_sc_agg(n_pad, d, cpt):
    rows_per_tile = n_pad // NS
    mesh = plsc.VectorSubcoreMesh(core_axis_name="c", subcore_axis_name="s")

    @functools.partial(
        pl.kernel,
        out_type=jax.ShapeDtypeStruct((NC, n_pad, d), jnp.float32),
        mesh=mesh,
        scratch_types=[
            pltpu.VMEM((cpt, K), jnp.int32),
            pltpu.VMEM((cpt, K), jnp.int32),
            pltpu.VMEM((cpt, K), jnp.float32),
            pltpu.VMEM((K, d), jnp.float32),
            pltpu.VMEM_SHARED((n_pad, d), jnp.float32),
            pltpu.SemaphoreType.DMA,
        ],
    )
    def sc_agg(pre_hbm, srcs_hbm, dsts_hbm, ws_hbm, zeros_hbm, out_hbm,
               src_v, dst_v, w_v, rows_v, acc_sh, sem):
        c = lax.axis_index("c")
        s = lax.axis_index("s")
        wid = c * NS + s

        pltpu.sync_copy(srcs_hbm.at[wid], src_v)
        pltpu.sync_copy(dsts_hbm.at[wid], dst_v)
        pltpu.sync_copy(ws_hbm.at[wid], w_v)

        sl = pl.ds(s * rows_per_tile, rows_per_tile)
        pltpu.sync_copy(zeros_hbm.at[sl], acc_sh.at[sl])
        plsc.subcore_barrier()

        def chunk_body(ci, _):
            pltpu.async_copy(pre_hbm.at[src_v.at[ci]], rows_v, sem).wait()

            def edge_body(e, _):
                ci_v = jnp.full((L,), ci, dtype=jnp.int32)
                e_v = jnp.full((L,), e, dtype=jnp.int32)
                wb = plsc.load_gather(w_v, [ci_v, e_v])
                for g in range(d // L):
                    cs = pl.ds(g * L, L)
                    rows_v[e, cs] = rows_v[e, cs] * wb
                return 0

            lax.fori_loop(0, K, edge_body, 0, unroll=2)

            pltpu.sync_copy(rows_v, acc_sh.at[dst_v.at[ci]], add=True)
            return 0

        lax.fori_loop(0, cpt, chunk_body, 0)
        plsc.subcore_barrier()

        pltpu.sync_copy(acc_sh.at[sl], out_hbm.at[c, sl])

    return sc_agg


def kernel(x, edge_index, edge_weight, W0, b):
    n, d_in = x.shape
    d_out = W0.shape[1]
    e = edge_index.shape[1]

    rb = 400
    grid = n // rb
    pre_sup = pl.pallas_call(
        _matmul_kernel,
        grid=(grid,),
        in_specs=[
            pl.BlockSpec((rb, d_in), lambda i: (i, 0)),
            pl.BlockSpec((d_in, d_out), lambda i: (0, 0)),
        ],
        out_specs=pl.BlockSpec((rb, d_out), lambda i: (i, 0)),
        out_shape=jax.ShapeDtypeStruct((n, d_out), jnp.float32),
    )(x, W0)

    per_tile = -(-e // NW)
    cpt = -(-per_tile // K)
    per_tile = cpt * K
    e_pad = per_tile * NW
    src = jnp.zeros((e_pad,), jnp.int32).at[:e].set(edge_index[0])
    dst = jnp.zeros((e_pad,), jnp.int32).at[:e].set(edge_index[1])
    w = jnp.zeros((e_pad,), jnp.float32).at[:e].set(edge_weight)
    srcs = src.reshape(NW, cpt, K)
    dsts = dst.reshape(NW, cpt, K)
    ws = w.reshape(NW, cpt, K)

    n_pad = -(-n // NS) * NS
    zeros = jnp.zeros((n_pad, d_out), jnp.float32)

    parts = _make_sc_agg(n_pad, d_out, cpt)(pre_sup, srcs, dsts, ws, zeros)

    p0 = parts[0, :n]
    p1 = parts[1, :n]
    out = pl.pallas_call(
        _combine_kernel,
        grid=(grid,),
        in_specs=[
            pl.BlockSpec((rb, d_out), lambda i: (i, 0)),
            pl.BlockSpec((rb, d_out), lambda i: (i, 0)),
            pl.BlockSpec((1, d_out), lambda i: (0, 0)),
        ],
        out_specs=pl.BlockSpec((rb, d_out), lambda i: (i, 0)),
        out_shape=jax.ShapeDtypeStruct((n, d_out), jnp.float32),
    )(p0, p1, b)
    return out


# speedup vs baseline: 4.0570x; 4.0570x over previous
import functools

import jax
import jax.numpy as jnp
from jax import lax
from jax.experimental import pallas as pl
from jax.experimental.pallas import tpu as pltpu
from jax.experimental.pallas import tpu_sc as plsc

NC = 2
NS = 16
NW = NC * NS
L = 16
K = 128


def _matmul_kernel(x_ref, w_ref, o_ref):
    o_ref[...] = jnp.dot(x_ref[...], w_ref[...],
                         preferred_element_type=jnp.float32)


def _combine_kernel(p0_ref, p1_ref, b_ref, o_ref):
    o_ref[...] = p0_ref[...] + p1_ref[...] + b_ref[...]


def _make_sc_agg(n_pad, d, cpt):
    rows_per_tile = n_pad // NS
    mesh = plsc.VectorSubcoreMesh(core_axis_name="c", subcore_axis_name="s")

    @functools.partial(
        pl.kernel,
        out_type=jax.ShapeDtypeStruct((NC, n_pad, d), jnp.float32),
        mesh=mesh,
        scratch_types=[
            pltpu.VMEM((cpt, K), jnp.int32),
            pltpu.VMEM((cpt, K), jnp.int32),
            pltpu.VMEM((cpt, K), jnp.float32),
            pltpu.VMEM((K, d), jnp.float32),
            pltpu.VMEM_SHARED((n_pad, d), jnp.float32),
            pltpu.SemaphoreType.DMA,
        ],
    )
    def sc_agg(pre_hbm, srcs_hbm, dsts_hbm, ws_hbm, zeros_hbm, out_hbm,
               src_v, dst_v, w_v, rows_v, acc_sh, sem):
        c = lax.axis_index("c")
        s = lax.axis_index("s")
        wid = c * NS + s

        pltpu.sync_copy(srcs_hbm.at[wid], src_v)
        pltpu.sync_copy(dsts_hbm.at[wid], dst_v)
        pltpu.sync_copy(ws_hbm.at[wid], w_v)

        sl = pl.ds(s * rows_per_tile, rows_per_tile)
        pltpu.sync_copy(zeros_hbm.at[sl], acc_sh.at[sl])
        plsc.subcore_barrier()

        def chunk_body(ci, _):
            pltpu.async_copy(pre_hbm.at[src_v.at[ci]], rows_v, sem).wait()

            def egroup_body(eg, _):
                wg = w_v[ci, pl.ds(eg * L, L)]
                for t in range(L):
                    wb = jnp.full((L,), wg[t], dtype=jnp.float32)
                    e = eg * L + t
                    for g in range(d // L):
                        cs = pl.ds(g * L, L)
                        rows_v[e, cs] = rows_v[e, cs] * wb
                return 0

            lax.fori_loop(0, K // L, egroup_body, 0)

            pltpu.sync_copy(rows_v, acc_sh.at[dst_v.at[ci]], add=True)
            return 0

        lax.fori_loop(0, cpt, chunk_body, 0)
        plsc.subcore_barrier()

        pltpu.sync_copy(acc_sh.at[sl], out_hbm.at[c, sl])

    return sc_agg


def kernel(x, edge_index, edge_weight, W0, b):
    n, d_in = x.shape
    d_out = W0.shape[1]
    e = edge_index.shape[1]

    rb = 400
    grid = n // rb
    pre_sup = pl.pallas_call(
        _matmul_kernel,
        grid=(grid,),
        in_specs=[
            pl.BlockSpec((rb, d_in), lambda i: (i, 0)),
            pl.BlockSpec((d_in, d_out), lambda i: (0, 0)),
        ],
        out_specs=pl.BlockSpec((rb, d_out), lambda i: (i, 0)),
        out_shape=jax.ShapeDtypeStruct((n, d_out), jnp.float32),
    )(x, W0)

    per_tile = -(-e // NW)
    cpt = -(-per_tile // K)
    per_tile = cpt * K
    e_pad = per_tile * NW
    src = jnp.zeros((e_pad,), jnp.int32).at[:e].set(edge_index[0])
    dst = jnp.zeros((e_pad,), jnp.int32).at[:e].set(edge_index[1])
    w = jnp.zeros((e_pad,), jnp.float32).at[:e].set(edge_weight)
    srcs = src.reshape(NW, cpt, K)
    dsts = dst.reshape(NW, cpt, K)
    ws = w.reshape(NW, cpt, K)

    n_pad = -(-n // (NS * 8)) * (NS * 8)
    zeros = jnp.zeros((n_pad, d_out), jnp.float32)

    parts = _make_sc_agg(n_pad, d_out, cpt)(pre_sup, srcs, dsts, ws, zeros)

    p0 = parts[0, :n]
    p1 = parts[1, :n]
    out = pl.pallas_call(
        _combine_kernel,
        grid=(grid,),
        in_specs=[
            pl.BlockSpec((rb, d_out), lambda i: (i, 0)),
            pl.BlockSpec((rb, d_out), lambda i: (i, 0)),
            pl.BlockSpec((1, d_out), lambda i: (0, 0)),
        ],
        out_specs=pl.BlockSpec((rb, d_out), lambda i: (i, 0)),
        out_shape=jax.ShapeDtypeStruct((n, d_out), jnp.float32),
    )(p0, p1, b)
    return out
